# baseline (speedup 1.0000x reference)
import functools

import jax
import jax.numpy as jnp
from jax import lax
from jax.experimental import pallas as pl
from jax.experimental.pallas import tpu as pltpu
from jax.experimental.pallas import tpu_sc as plsc

_TOPK = 2
_E = 16
_NC = 2
_NS = 16
_NW = _NC * _NS


def _logits_kernel(x_ref, w_ref, lt_ref):
    xb = x_ref[...]
    w = w_ref[...]
    logits = jax.lax.dot_general(
        xb, w, (((1,), (1,)), ((), ())),
        preferred_element_type=jnp.float32)
    lt_ref[...] = logits.T


@functools.partial(jax.jit, static_argnames=("blk",))
def _logits_t(x2, w, blk):
    n, d = x2.shape
    e = w.shape[0]
    grid = n // blk
    return pl.pallas_call(
        _logits_kernel,
        grid=(grid,),
        in_specs=[
            pl.BlockSpec((blk, d), lambda i: (i, 0)),
            pl.BlockSpec((e, d), lambda i: (0, 0)),
        ],
        out_specs=pl.BlockSpec((e, blk), lambda i: (0, i)),
        out_shape=jax.ShapeDtypeStruct((e, n), jnp.float32),
    )(x2, w)


def _make_router(n):
    tpw = n // _NW
    ngrp = tpw // 16
    mesh = plsc.VectorSubcoreMesh(core_axis_name="c", subcore_axis_name="s")

    @functools.partial(
        pl.kernel, mesh=mesh,
        out_type=[
            jax.ShapeDtypeStruct((_TOPK, n), jnp.int32),
            jax.ShapeDtypeStruct((_TOPK, n), jnp.float32),
        ],
        scratch_types=[
            pltpu.VMEM((_E, tpw), jnp.float32),
            pltpu.VMEM((_TOPK, tpw), jnp.int32),
            pltpu.VMEM((_TOPK, tpw), jnp.float32),
        ],
    )
    def route(lt_hbm, idx_hbm, wgt_hbm, lv, iv, wv):
        wid = lax.axis_index("s") * _NC + lax.axis_index("c")
        base = wid * tpw
        pltpu.sync_copy(lt_hbm.at[:, pl.ds(base * 1, tpw)], lv)

        def group(g, carry):
            off = g * 16
            ge = [lv[e, pl.ds(off, 16)] for e in range(_E)]
            m1 = ge[0]
            i1 = jnp.zeros((16,), jnp.int32)
            for e in range(1, _E):
                gt = ge[e] > m1
                m1 = jnp.where(gt, ge[e], m1)
                i1 = jnp.where(gt, jnp.full((16,), e, jnp.int32), i1)
            m2 = jnp.full((16,), -3.0e38, jnp.float32)
            i2 = jnp.full((16,), _E, jnp.int32)
            z = jnp.zeros((16,), jnp.float32)
            for e in range(_E):
                z = z + jnp.exp(ge[e] - m1)
                ev = jnp.full((16,), e, jnp.int32)
                sel = jnp.logical_and(ev != i1, ge[e] > m2)
                m2 = jnp.where(sel, ge[e], m2)
                i2 = jnp.where(sel, ev, i2)
            t = jnp.exp(m2 - m1)
            denom = 1.0 + t + 1e-9 * z
            iv[0, pl.ds(off, 16)] = i1
            iv[1, pl.ds(off, 16)] = i2
            wv[0, pl.ds(off, 16)] = 1.0 / denom
            wv[1, pl.ds(off, 16)] = t / denom
            return carry

        lax.fori_loop(0, ngrp, group, 0)
        pltpu.sync_copy(iv, idx_hbm.at[:, pl.ds(base * 1, tpw)])
        pltpu.sync_copy(wv, wgt_hbm.at[:, pl.ds(base * 1, tpw)])

    return route


@jax.jit
def _gate_sc(x2, w):
    n, d = x2.shape
    lt = _logits_t(x2, w, 1024)
    idx2, wgt2 = _make_router(n)(lt)
    return idx2.T, wgt2.T


def kernel(x, W):
    b, s, d = x.shape
    x2 = x.reshape(b * s, d)
    return _gate_sc(x2, W)


# speedup vs baseline: 1.4624x; 1.4624x over previous
import functools

import jax
import jax.numpy as jnp
from jax.experimental import pallas as pl

_TOPK = 2
_NEG_INF = float("-inf")


def _gate_kernel(x_ref, w_ref, idx_ref, wgt_ref):
    xb = x_ref[...]
    w = w_ref[...]
    logits = jax.lax.dot_general(
        xb, w, (((1,), (1,)), ((), ())),
        preferred_element_type=jnp.float32)

    lt = logits.T
    e, blk = lt.shape
    row = jax.lax.broadcasted_iota(jnp.int32, (e, blk), 0)

    m = jnp.max(lt, axis=0, keepdims=True)
    ev = jnp.exp(lt - m)
    s = ev / jnp.sum(ev, axis=0, keepdims=True)

    s1 = jnp.max(s, axis=0, keepdims=True)
    i1 = jnp.min(jnp.where(s == s1, row, e), axis=0, keepdims=True)
    masked = jnp.where(row == i1, -1.0, s)
    s2 = jnp.max(masked, axis=0, keepdims=True)
    i2 = jnp.min(jnp.where(masked == s2, row, e), axis=0, keepdims=True)

    denom = s1 + s2 + 1e-9
    w1 = s1 / denom
    w2 = s2 / denom

    idx = jnp.concatenate([i1, i2], axis=0).astype(jnp.int32)
    wgt = jnp.concatenate([w1, w2], axis=0)
    idx_ref[...] = idx[None]
    wgt_ref[...] = wgt[None]


@functools.partial(jax.jit, static_argnames=("blk",))
def _gate(x2, w, blk):
    n, d = x2.shape
    e = w.shape[0]
    grid = n // blk
    idx, wgt = pl.pallas_call(
        _gate_kernel,
        grid=(grid,),
        in_specs=[
            pl.BlockSpec((blk, d), lambda i: (i, 0)),
            pl.BlockSpec((e, d), lambda i: (0, 0)),
        ],
        out_specs=[
            pl.BlockSpec((1, _TOPK, blk), lambda i: (i, 0, 0)),
            pl.BlockSpec((1, _TOPK, blk), lambda i: (i, 0, 0)),
        ],
        out_shape=[
            jax.ShapeDtypeStruct((grid, _TOPK, blk), jnp.int32),
            jax.ShapeDtypeStruct((grid, _TOPK, blk), jnp.float32),
        ],
    )(x2, w)
    idx = idx.transpose(0, 2, 1).reshape(n, _TOPK)
    wgt = wgt.transpose(0, 2, 1).reshape(n, _TOPK)
    return idx, wgt


def kernel(x, W):
    b, s, d = x.shape
    x2 = x.reshape(b * s, d)
    return _gate(x2, W, 1024)


# speedup vs baseline: 1.4636x; 1.0008x over previous
import functools

import jax
import jax.numpy as jnp
from jax.experimental import pallas as pl

_TOPK = 2


def _gate_kernel(x_ref, w_ref, idx_ref, wgt_ref):
    xb = x_ref[...]
    w = w_ref[...]
    logits = jax.lax.dot_general(
        xb, w, (((1,), (1,)), ((), ())),
        preferred_element_type=jnp.float32)

    lt = logits.T
    e, blk = lt.shape
    row = jax.lax.broadcasted_iota(jnp.int32, (e, blk), 0)

    m = jnp.max(lt, axis=0, keepdims=True)
    ev = jnp.exp(lt - m)
    s = ev / jnp.sum(ev, axis=0, keepdims=True)

    s1 = jnp.max(s, axis=0, keepdims=True)
    i1 = jnp.min(jnp.where(s == s1, row, e), axis=0, keepdims=True)
    masked = jnp.where(row == i1, -1.0, s)
    s2 = jnp.max(masked, axis=0, keepdims=True)
    i2 = jnp.min(jnp.where(masked == s2, row, e), axis=0, keepdims=True)

    denom = s1 + s2 + 1e-9
    w1 = s1 / denom
    w2 = s2 / denom

    idx = jnp.concatenate([i1, i2], axis=0).astype(jnp.int32)
    wgt = jnp.concatenate([w1, w2], axis=0)
    idx_ref[...] = idx[None]
    wgt_ref[...] = wgt[None]


@functools.partial(jax.jit, static_argnames=("blk",))
def _gate(x2, w, blk):
    n, d = x2.shape
    e = w.shape[0]
    grid = n // blk
    idx, wgt = pl.pallas_call(
        _gate_kernel,
        grid=(grid,),
        in_specs=[
            pl.BlockSpec((blk, d), lambda i: (i, 0)),
            pl.BlockSpec((e, d), lambda i: (0, 0)),
        ],
        out_specs=[
            pl.BlockSpec((1, _TOPK, blk), lambda i: (i, 0, 0)),
            pl.BlockSpec((1, _TOPK, blk), lambda i: (i, 0, 0)),
        ],
        out_shape=[
            jax.ShapeDtypeStruct((grid, _TOPK, blk), jnp.int32),
            jax.ShapeDtypeStruct((grid, _TOPK, blk), jnp.float32),
        ],
    )(x2, w)
    idx = idx.transpose(0, 2, 1).reshape(n, _TOPK)
    wgt = wgt.transpose(0, 2, 1).reshape(n, _TOPK)
    return idx, wgt


def kernel(x, W):
    b, s, d = x.shape
    x2 = x.reshape(b * s, d)
    return _gate(x2, W, 1024)
